# Initial kernel scaffold; baseline (speedup 1.0000x reference)
import jax
import jax.numpy as jnp
from jax.experimental import pallas as pl


def kernel(x, W_ext, b_ext, W_conv1, b_conv1, W_loc, b_loc, W_score, b_score):
    raise NotImplementedError("write your pallas kernel here")


# speedup vs baseline: 3.2981x; 3.2981x over previous
import jax
import jax.numpy as jnp
from jax.experimental import pallas as pl

_S = 16
_H = 50
_W = 50
_K = 768
_O = 512
_NPAD = 2560
_BLK_N = 512


def _mm_kernel(w_ref, p_ref, b_ref, o_ref):
    o_ref[...] = (
        jnp.dot(w_ref[...], p_ref[...], preferred_element_type=jnp.float32)
        + b_ref[...]
    )


def kernel(x, W_ext, b_ext, W_conv1, b_conv1, W_loc, b_loc, W_score, b_score):
    patches = (
        x[0]
        .reshape(3, _H, _S, _W, _S)
        .transpose(0, 2, 4, 1, 3)
        .reshape(_K, _H * _W)
    )
    patches = jnp.pad(patches, ((0, 0), (0, _NPAD - _H * _W)))
    w_flat = W_ext.reshape(_O, _K)
    bias = b_ext.reshape(_O, 1)

    out = pl.pallas_call(
        _mm_kernel,
        grid=(_NPAD // _BLK_N,),
        in_specs=[
            pl.BlockSpec((_O, _K), lambda n: (0, 0)),
            pl.BlockSpec((_K, _BLK_N), lambda n: (0, n)),
            pl.BlockSpec((_O, 1), lambda n: (0, 0)),
        ],
        out_specs=pl.BlockSpec((_O, _BLK_N), lambda n: (0, n)),
        out_shape=jax.ShapeDtypeStruct((_O, _NPAD), jnp.float32),
    )(w_flat, patches, bias)

    return out[:, : _H * _W].reshape(1, _O, _H, _W)


# speedup vs baseline: 4.1160x; 1.2480x over previous
import jax
import jax.numpy as jnp
from jax.experimental import pallas as pl

_S = 16
_H = 50
_W = 50
_N = _H * _W
_K = 768
_O = 512
_BLK_N = 512


def _mm_kernel(w_ref, p_ref, b_ref, o_ref):
    o_ref[...] = (
        jnp.dot(w_ref[...], p_ref[...], preferred_element_type=jnp.float32)
        + b_ref[...]
    )


def kernel(x, W_ext, b_ext, W_conv1, b_conv1, W_loc, b_loc, W_score, b_score):
    patches = (
        x[0]
        .reshape(3, _H, _S, _W, _S)
        .transpose(0, 2, 4, 1, 3)
        .reshape(_K, _N)
        .astype(jnp.bfloat16)
    )
    w_flat = W_ext.reshape(_O, _K).astype(jnp.bfloat16)
    bias = b_ext.reshape(_O, 1)

    out = pl.pallas_call(
        _mm_kernel,
        grid=(pl.cdiv(_N, _BLK_N),),
        in_specs=[
            pl.BlockSpec((_O, _K), lambda n: (0, 0)),
            pl.BlockSpec((_K, _BLK_N), lambda n: (0, n)),
            pl.BlockSpec((_O, 1), lambda n: (0, 0)),
        ],
        out_specs=pl.BlockSpec((_O, _BLK_N), lambda n: (0, n)),
        out_shape=jax.ShapeDtypeStruct((_O, _N), jnp.float32),
    )(w_flat, patches, bias)

    return out.reshape(1, _O, _H, _W)
